# Initial kernel scaffold; baseline (speedup 1.0000x reference)
import jax
import jax.numpy as jnp
from jax.experimental import pallas as pl


def kernel(x, edge_index, edge_attr, params):
    raise NotImplementedError("write your pallas kernel here")


# speedup vs baseline: 23.5837x; 23.5837x over previous
import functools

import jax
import jax.numpy as jnp
import numpy as np
from jax import lax
from jax.experimental import pallas as pl
from jax.experimental.pallas import tpu as pltpu
from jax.experimental.pallas import tpu_sc as plsc

N = 10000
E = 160000
D = 256
H = 8
DH = 32

NPAD = 10240
NC = 2
NS = 16
NW = NC * NS
SCCH = 128

BM_E = 1000
BM_N = 1000


def _gather_rows(table, idx, dt):
    etot = idx.shape[0]
    nchunks = etot // SCCH
    nfull = nchunks // NW
    rem = nchunks % NW
    mesh = plsc.VectorSubcoreMesh(core_axis_name="c", subcore_axis_name="s")

    @functools.partial(
        pl.kernel,
        mesh=mesh,
        out_type=jax.ShapeDtypeStruct((etot, dt), jnp.float32),
        scratch_types=[
            pltpu.VMEM((SCCH,), jnp.int32),
            pltpu.VMEM((SCCH, dt), jnp.float32),
            pltpu.SemaphoreType.DMA,
        ],
    )
    def k(table_hbm, idx_hbm, out_hbm, idx_v, rows_v, sem):
        wid = lax.axis_index("s") * NC + lax.axis_index("c")

        def body(j, carry):
            base = (j * NW + wid) * SCCH
            pltpu.sync_copy(idx_hbm.at[pl.ds(base, SCCH)], idx_v)
            pltpu.async_copy(table_hbm.at[idx_v], rows_v, sem).wait()
            pltpu.sync_copy(rows_v, out_hbm.at[pl.ds(base, SCCH)])
            return carry

        lax.fori_loop(0, nfull, body, 0)

        @pl.when(wid < rem)
        def _():
            body(nfull, 0)

    return k(table, idx)


def _scatter_add(vals, idx, zeros, c):
    nchunks = E // SCCH
    nfull = nchunks // NW
    rem = nchunks % NW
    rows_per_tile = NPAD // NS
    mesh = plsc.VectorSubcoreMesh(core_axis_name="c", subcore_axis_name="s")

    @functools.partial(
        pl.kernel,
        mesh=mesh,
        out_type=jax.ShapeDtypeStruct((NC, NPAD, c), jnp.float32),
        scratch_types=[
            pltpu.VMEM((SCCH,), jnp.int32),
            pltpu.VMEM((SCCH, c), jnp.float32),
            pltpu.VMEM_SHARED((NPAD, c), jnp.float32),
        ],
    )
    def k(vals_hbm, idx_hbm, zeros_hbm, out_hbm, idx_v, val_v, acc):
        ci = lax.axis_index("c")
        si = lax.axis_index("s")
        wid = si * NC + ci
        r0 = si * rows_per_tile
        pltpu.sync_copy(zeros_hbm, acc.at[pl.ds(r0, rows_per_tile)])
        plsc.subcore_barrier()

        def body(j, carry):
            base = (j * NW + wid) * SCCH
            pltpu.sync_copy(idx_hbm.at[pl.ds(base, SCCH)], idx_v)
            pltpu.sync_copy(vals_hbm.at[pl.ds(base, SCCH)], val_v)
            pltpu.sync_copy(val_v, acc.at[idx_v], add=True)
            return carry

        lax.fori_loop(0, nfull, body, 0)

        @pl.when(wid < rem)
        def _():
            body(nfull, 0)

        plsc.subcore_barrier()
        pltpu.sync_copy(acc.at[pl.ds(r0, rows_per_tile)],
                        out_hbm.at[ci, pl.ds(r0, rows_per_tile)])

    return k(vals, idx, zeros)


def _mm_bias(x, w, b, bm, extra=None):
    m, kdim = x.shape
    nc = w.shape[1]
    grid = (m // bm,)

    def body_noextra(x_ref, w_ref, b_ref, o_ref):
        o_ref[...] = (jnp.dot(x_ref[...], w_ref[...],
                              preferred_element_type=jnp.float32) + b_ref[...])

    def body_extra(x_ref, w_ref, b_ref, e_ref, o_ref):
        o_ref[...] = (jnp.dot(x_ref[...], w_ref[...],
                              preferred_element_type=jnp.float32)
                      + b_ref[...] + e_ref[...])

    in_specs = [
        pl.BlockSpec((bm, kdim), lambda i: (i, 0)),
        pl.BlockSpec((kdim, nc), lambda i: (0, 0)),
        pl.BlockSpec((1, nc), lambda i: (0, 0)),
    ]
    args = [x, w, b]
    body = body_noextra
    if extra is not None:
        in_specs.append(pl.BlockSpec((bm, nc), lambda i: (i, 0)))
        args.append(extra)
        body = body_extra
    return pl.pallas_call(
        body,
        grid=grid,
        in_specs=in_specs,
        out_specs=pl.BlockSpec((bm, nc), lambda i: (i, 0)),
        out_shape=jax.ShapeDtypeStruct((m, nc), jnp.float32),
    )(*args)


def _qkv_proj(x, wqkv, bqkv):
    grid = (N // BM_N,)

    def body(x_ref, w_ref, b_ref, q_ref, kv_ref):
        y = (jnp.dot(x_ref[...], w_ref[...],
                     preferred_element_type=jnp.float32) + b_ref[...])
        q_ref[...] = y[:, :D]
        kv_ref[...] = y[:, D:]

    return pl.pallas_call(
        body,
        grid=grid,
        in_specs=[
            pl.BlockSpec((BM_N, D), lambda i: (i, 0)),
            pl.BlockSpec((D, 3 * D), lambda i: (0, 0)),
            pl.BlockSpec((1, 3 * D), lambda i: (0, 0)),
        ],
        out_specs=[
            pl.BlockSpec((BM_N, D), lambda i: (i, 0)),
            pl.BlockSpec((BM_N, 2 * D), lambda i: (i, 0)),
        ],
        out_shape=[
            jax.ShapeDtypeStruct((N, D), jnp.float32),
            jax.ShapeDtypeStruct((N, 2 * D), jnp.float32),
        ],
    )(x, wqkv, bqkv)


def _edge_stage(kvg, qg, eweb, awmat, rep):
    grid = (E // BM_E,)

    def body(kv_ref, q_ref, e_ref, aw_ref, rep_ref,
             et_ref, vplo_ref, vphi_ref, etplo_ref, etphi_ref, p16_ref):
        kv = kv_ref[...]
        kk = kv[:, :D]
        vv = kv[:, D:]
        ew = e_ref[:, :D]
        eb = e_ref[:, D:]
        sc = (kk + q_ref[...]) * ew
        sc = jnp.sign(sc) * jnp.sqrt(jnp.abs(sc)) + eb
        et = jnp.maximum(sc, 0.0)
        et_ref[...] = et
        s = jnp.dot(et, aw_ref[...], preferred_element_type=jnp.float32)
        s = jnp.clip(s, -5.0, 5.0)
        p = jnp.exp(s - 5.0)
        p256 = jnp.dot(p, rep_ref[...], preferred_element_type=jnp.float32)
        vp = vv * p256
        etp = et * p256
        vplo_ref[...] = vp[:, :128]
        vphi_ref[...] = vp[:, 128:]
        etplo_ref[...] = etp[:, :128]
        etphi_ref[...] = etp[:, 128:]
        p16_ref[...] = jnp.concatenate(
            [p, jnp.ones_like(p), jnp.zeros((p.shape[0], 112), jnp.float32)],
            axis=1)

    outs = pl.pallas_call(
        body,
        grid=grid,
        in_specs=[
            pl.BlockSpec((BM_E, 2 * D), lambda i: (i, 0)),
            pl.BlockSpec((BM_E, D), lambda i: (i, 0)),
            pl.BlockSpec((BM_E, 2 * D), lambda i: (i, 0)),
            pl.BlockSpec((D, H), lambda i: (0, 0)),
            pl.BlockSpec((H, D), lambda i: (0, 0)),
        ],
        out_specs=[
            pl.BlockSpec((BM_E, D), lambda i: (i, 0)),
            pl.BlockSpec((BM_E, 128), lambda i: (i, 0)),
            pl.BlockSpec((BM_E, 128), lambda i: (i, 0)),
            pl.BlockSpec((BM_E, 128), lambda i: (i, 0)),
            pl.BlockSpec((BM_E, 128), lambda i: (i, 0)),
            pl.BlockSpec((BM_E, 128), lambda i: (i, 0)),
        ],
        out_shape=[
            jax.ShapeDtypeStruct((E, D), jnp.float32),
            jax.ShapeDtypeStruct((E, 128), jnp.float32),
            jax.ShapeDtypeStruct((E, 128), jnp.float32),
            jax.ShapeDtypeStruct((E, 128), jnp.float32),
            jax.ShapeDtypeStruct((E, 128), jnp.float32),
            jax.ShapeDtypeStruct((E, 128), jnp.float32),
        ],
    )(kvg, qg, eweb, awmat, rep)
    return outs


def _node_combine(vplo, vphi, etplo, etphi, p16, x, rep, vem, cc, woh, boh):
    grid = (N // BM_N,)

    def body(vplo_ref, vphi_ref, etplo_ref, etphi_ref, p16_ref, x_ref,
             rep_ref, vem_ref, cc_ref, woh_ref, boh_ref, t_ref):
        a = jnp.concatenate([vplo_ref[0] + vplo_ref[1],
                             vphi_ref[0] + vphi_ref[1]], axis=1)
        bm = jnp.concatenate([etplo_ref[0] + etplo_ref[1],
                              etphi_ref[0] + etphi_ref[1]], axis=1)
        pp = p16_ref[0] + p16_ref[1]
        ssum = pp[:, :H]
        deg = pp[:, H:H + 1]
        denom = jnp.dot(ssum, rep_ref[...],
                        preferred_element_type=jnp.float32) + 1e-16
        wv = a / denom
        rowv = jnp.dot(bm / denom, vem_ref[...],
                       preferred_element_type=jnp.float32)
        h1 = wv + rowv
        ld = jnp.log(deg + 1.0)
        h2 = h1 * cc_ref[0:1, :] + (h1 * ld) * cc_ref[1:2, :]
        t_ref[...] = (x_ref[...]
                      + jnp.dot(h2, woh_ref[...],
                                preferred_element_type=jnp.float32)
                      + boh_ref[...])

    return pl.pallas_call(
        body,
        grid=grid,
        in_specs=[
            pl.BlockSpec((2, BM_N, 128), lambda i: (0, i, 0)),
            pl.BlockSpec((2, BM_N, 128), lambda i: (0, i, 0)),
            pl.BlockSpec((2, BM_N, 128), lambda i: (0, i, 0)),
            pl.BlockSpec((2, BM_N, 128), lambda i: (0, i, 0)),
            pl.BlockSpec((2, BM_N, 128), lambda i: (0, i, 0)),
            pl.BlockSpec((BM_N, D), lambda i: (i, 0)),
            pl.BlockSpec((H, D), lambda i: (0, 0)),
            pl.BlockSpec((D, D), lambda i: (0, 0)),
            pl.BlockSpec((2, D), lambda i: (0, 0)),
            pl.BlockSpec((D, D), lambda i: (0, 0)),
            pl.BlockSpec((1, D), lambda i: (0, 0)),
        ],
        out_specs=pl.BlockSpec((BM_N, D), lambda i: (i, 0)),
        out_shape=jax.ShapeDtypeStruct((N, D), jnp.float32),
    )(vplo, vphi, etplo, etphi, p16, x, rep, vem, cc, woh, boh)


def _bn_reduce(x, bm):
    m, cols = x.shape
    grid = (m // bm,)

    def body(x_ref, s_ref):
        i = pl.program_id(0)

        @pl.when(i == 0)
        def _():
            s_ref[...] = jnp.zeros_like(s_ref)

        xv = x_ref[...]
        upd = jnp.concatenate(
            [jnp.sum(xv, axis=0, keepdims=True),
             jnp.sum(xv * xv, axis=0, keepdims=True),
             jnp.zeros((6, cols), jnp.float32)], axis=0)
        s_ref[...] += upd

    return pl.pallas_call(
        body,
        grid=grid,
        in_specs=[pl.BlockSpec((bm, cols), lambda i: (i, 0))],
        out_specs=pl.BlockSpec((8, cols), lambda i: (0, 0)),
        out_shape=jax.ShapeDtypeStruct((8, cols), jnp.float32),
    )(x)


def _bn_apply(x, stats, gb, bm, nrows):
    m, cols = x.shape
    grid = (m // bm,)
    inv_n = 1.0 / nrows

    def body(x_ref, s_ref, gb_ref, o_ref):
        s = s_ref[...]
        mu = s[0:1, :] * inv_n
        var = s[1:2, :] * inv_n - mu * mu
        inv = lax.rsqrt(var + 1e-5)
        o_ref[...] = gb_ref[0:1, :] * (x_ref[...] - mu) * inv + gb_ref[1:2, :]

    return pl.pallas_call(
        body,
        grid=grid,
        in_specs=[
            pl.BlockSpec((bm, cols), lambda i: (i, 0)),
            pl.BlockSpec((8, cols), lambda i: (0, 0)),
            pl.BlockSpec((2, cols), lambda i: (0, 0)),
        ],
        out_specs=pl.BlockSpec((bm, cols), lambda i: (i, 0)),
        out_shape=jax.ShapeDtypeStruct((m, cols), jnp.float32),
    )(x, stats, gb)


def _ffn_stage(t, stats, gb1, w1, b1, w2, b2):
    grid = (N // BM_N,)
    inv_n = 1.0 / N

    def body(t_ref, s_ref, gb_ref, w1_ref, b1_ref, w2_ref, b2_ref,
             h3_ref, s2_ref):
        i = pl.program_id(0)
        s = s_ref[...]
        mu = s[0:1, :] * inv_n
        var = s[1:2, :] * inv_n - mu * mu
        inv = lax.rsqrt(var + 1e-5)
        hb = gb_ref[0:1, :] * (t_ref[...] - mu) * inv + gb_ref[1:2, :]
        f = jnp.maximum(
            jnp.dot(hb, w1_ref[...], preferred_element_type=jnp.float32)
            + b1_ref[...], 0.0)
        h3 = hb + (jnp.dot(f, w2_ref[...], preferred_element_type=jnp.float32)
                   + b2_ref[...])
        h3_ref[...] = h3

        @pl.when(i == 0)
        def _():
            s2_ref[...] = jnp.zeros_like(s2_ref)

        s2_ref[...] += jnp.concatenate(
            [jnp.sum(h3, axis=0, keepdims=True),
             jnp.sum(h3 * h3, axis=0, keepdims=True),
             jnp.zeros((6, D), jnp.float32)], axis=0)

    return pl.pallas_call(
        body,
        grid=grid,
        in_specs=[
            pl.BlockSpec((BM_N, D), lambda i: (i, 0)),
            pl.BlockSpec((8, D), lambda i: (0, 0)),
            pl.BlockSpec((2, D), lambda i: (0, 0)),
            pl.BlockSpec((D, 2 * D), lambda i: (0, 0)),
            pl.BlockSpec((1, 2 * D), lambda i: (0, 0)),
            pl.BlockSpec((2 * D, D), lambda i: (0, 0)),
            pl.BlockSpec((1, D), lambda i: (0, 0)),
        ],
        out_specs=[
            pl.BlockSpec((BM_N, D), lambda i: (i, 0)),
            pl.BlockSpec((8, D), lambda i: (0, 0)),
        ],
        out_shape=[
            jax.ShapeDtypeStruct((N, D), jnp.float32),
            jax.ShapeDtypeStruct((8, D), jnp.float32),
        ],
    )(t, stats, gb1, w1, b1, w2, b2)


_WE_PERM = np.array(
    [h * 2 * DH + j for h in range(H) for j in range(DH)]
    + [h * 2 * DH + DH + j for h in range(H) for j in range(DH)],
    dtype=np.int32)

_REP = np.zeros((H, D), np.float32)
for _h in range(H):
    _REP[_h, _h * DH:(_h + 1) * DH] = 1.0

_HEAD_MASK = (np.arange(D)[:, None] // DH == np.arange(H)[None, :])


def kernel(x, edge_index, edge_attr, params):
    src = edge_index[0]
    dst = edge_index[1]

    wqkv = jnp.concatenate([params['Wq'], params['Wk'], params['Wv']], axis=1)
    bqkv = jnp.concatenate(
        [params['bq'], jnp.zeros((2 * H * DH,), jnp.float32)])[None, :]
    wep = params['We'][:, _WE_PERM]
    bep = params['be'][_WE_PERM][None, :]
    rep = jnp.asarray(_REP)
    aw2 = params['Aw'][:, :, 0]
    awmat = jnp.where(jnp.asarray(_HEAD_MASK),
                      jnp.tile(aw2, (H, 1)), 0.0)
    vem = jax.scipy.linalg.block_diag(
        *[params['VeRow'][:, h, :] for h in range(H)])
    cc = params['deg_coef'][0].T
    gb1h = jnp.stack([params['g1h'], params['b1h']])
    gb1e = jnp.stack([params['g1e'], params['b1e']])
    gb2h = jnp.stack([params['g2h'], params['b2h']])
    zeros128 = jnp.zeros((NPAD // NS, 128), jnp.float32)

    qt, kvt = _qkv_proj(x, wqkv, bqkv)
    eweb = _mm_bias(edge_attr, wep, bep, BM_E)

    kvg = _gather_rows(kvt, src, 2 * D)
    qg = _gather_rows(qt, dst, D)

    et, vplo, vphi, etplo, etphi, p16 = _edge_stage(kvg, qg, eweb, awmat, rep)

    a_lo = _scatter_add(vplo, dst, zeros128, 128)[:, :N, :]
    a_hi = _scatter_add(vphi, dst, zeros128, 128)[:, :N, :]
    b_lo = _scatter_add(etplo, dst, zeros128, 128)[:, :N, :]
    b_hi = _scatter_add(etphi, dst, zeros128, 128)[:, :N, :]
    pacc = _scatter_add(p16, dst, zeros128, 128)[:, :N, :]

    t = _node_combine(a_lo, a_hi, b_lo, b_hi, pacc, x, rep, vem, cc,
                      params['Woh'], params['boh'][None, :])

    stats1 = _bn_reduce(t, BM_N)
    h3, stats2 = _ffn_stage(t, stats1, gb1h, params['W1'],
                            params['bf1'][None, :], params['W2'],
                            params['bf2'][None, :])
    h_out = _bn_apply(h3, stats2, gb2h, BM_N, N)

    ee_pre = _mm_bias(et, params['Woe'], params['boe'][None, :], BM_E,
                      extra=edge_attr)
    stats_e = _bn_reduce(ee_pre, BM_E)
    ee_out = _bn_apply(ee_pre, stats_e, gb1e, BM_E, E)

    return h_out, ee_out


# speedup vs baseline: 26.5606x; 1.1262x over previous
import functools

import jax
import jax.numpy as jnp
import numpy as np
from jax import lax
from jax.experimental import pallas as pl
from jax.experimental.pallas import tpu as pltpu
from jax.experimental.pallas import tpu_sc as plsc

N = 10000
E = 160000
D = 256
H = 8
DH = 32

NPAD = 10240
NC = 2
NS = 16
NW = NC * NS
SCCH = 128

BM_E = 1000
BM_N = 1000


def _gather_rows(table, idx, dt):
    etot = idx.shape[0]
    nchunks = etot // SCCH
    nfull = nchunks // NW
    rem = nchunks % NW
    mesh = plsc.VectorSubcoreMesh(core_axis_name="c", subcore_axis_name="s")

    @functools.partial(
        pl.kernel,
        mesh=mesh,
        out_type=jax.ShapeDtypeStruct((etot, dt), jnp.float32),
        scratch_types=[
            pltpu.VMEM((SCCH,), jnp.int32),
            pltpu.VMEM((SCCH, dt), jnp.float32),
            pltpu.SemaphoreType.DMA,
        ],
    )
    def k(table_hbm, idx_hbm, out_hbm, idx_v, rows_v, sem):
        wid = lax.axis_index("s") * NC + lax.axis_index("c")

        def body(j, carry):
            base = (j * NW + wid) * SCCH
            pltpu.sync_copy(idx_hbm.at[pl.ds(base, SCCH)], idx_v)
            pltpu.async_copy(table_hbm.at[idx_v], rows_v, sem).wait()
            pltpu.sync_copy(rows_v, out_hbm.at[pl.ds(base, SCCH)])
            return carry

        lax.fori_loop(0, nfull, body, 0)

        @pl.when(wid < rem)
        def _():
            body(nfull, 0)

    return k(table, idx)


def _scatter_add(vals, idx, zeros, c):
    nchunks = E // SCCH
    nfull = nchunks // NW
    rem = nchunks % NW
    rows_per_tile = NPAD // NS
    mesh = plsc.VectorSubcoreMesh(core_axis_name="c", subcore_axis_name="s")

    @functools.partial(
        pl.kernel,
        mesh=mesh,
        out_type=jax.ShapeDtypeStruct((NC, NPAD, c), jnp.float32),
        scratch_types=[
            pltpu.VMEM((SCCH,), jnp.int32),
            pltpu.VMEM((SCCH, c), jnp.float32),
            pltpu.VMEM_SHARED((NPAD, c), jnp.float32),
        ],
    )
    def k(vals_hbm, idx_hbm, zeros_hbm, out_hbm, idx_v, val_v, acc):
        ci = lax.axis_index("c")
        si = lax.axis_index("s")
        wid = si * NC + ci
        r0 = si * rows_per_tile
        pltpu.sync_copy(zeros_hbm, acc.at[pl.ds(r0, rows_per_tile)])
        plsc.subcore_barrier()

        def body(j, carry):
            base = (j * NW + wid) * SCCH
            pltpu.sync_copy(idx_hbm.at[pl.ds(base, SCCH)], idx_v)
            pltpu.sync_copy(vals_hbm.at[pl.ds(base, SCCH)], val_v)
            pltpu.sync_copy(val_v, acc.at[idx_v], add=True)
            return carry

        lax.fori_loop(0, nfull, body, 0)

        @pl.when(wid < rem)
        def _():
            body(nfull, 0)

        plsc.subcore_barrier()
        pltpu.sync_copy(acc.at[pl.ds(r0, rows_per_tile)],
                        out_hbm.at[ci, pl.ds(r0, rows_per_tile)])

    return k(vals, idx, zeros)


def _mm_bias(x, w, b, bm, extra=None):
    m, kdim = x.shape
    nc = w.shape[1]
    grid = (m // bm,)

    def body_noextra(x_ref, w_ref, b_ref, o_ref):
        o_ref[...] = (jnp.dot(x_ref[...], w_ref[...],
                              preferred_element_type=jnp.float32) + b_ref[...])

    def body_extra(x_ref, w_ref, b_ref, e_ref, o_ref):
        o_ref[...] = (jnp.dot(x_ref[...], w_ref[...],
                              preferred_element_type=jnp.float32)
                      + b_ref[...] + e_ref[...])

    in_specs = [
        pl.BlockSpec((bm, kdim), lambda i: (i, 0)),
        pl.BlockSpec((kdim, nc), lambda i: (0, 0)),
        pl.BlockSpec((1, nc), lambda i: (0, 0)),
    ]
    args = [x, w, b]
    body = body_noextra
    if extra is not None:
        in_specs.append(pl.BlockSpec((bm, nc), lambda i: (i, 0)))
        args.append(extra)
        body = body_extra
    return pl.pallas_call(
        body,
        grid=grid,
        in_specs=in_specs,
        out_specs=pl.BlockSpec((bm, nc), lambda i: (i, 0)),
        out_shape=jax.ShapeDtypeStruct((m, nc), jnp.float32),
    )(*args)


def _qkv_proj(x, wqkv, bqkv):
    grid = (N // BM_N,)

    def body(x_ref, w_ref, b_ref, q_ref, kv_ref):
        y = (jnp.dot(x_ref[...], w_ref[...],
                     preferred_element_type=jnp.float32) + b_ref[...])
        q_ref[...] = y[:, :D]
        kv_ref[...] = y[:, D:]

    return pl.pallas_call(
        body,
        grid=grid,
        in_specs=[
            pl.BlockSpec((BM_N, D), lambda i: (i, 0)),
            pl.BlockSpec((D, 3 * D), lambda i: (0, 0)),
            pl.BlockSpec((1, 3 * D), lambda i: (0, 0)),
        ],
        out_specs=[
            pl.BlockSpec((BM_N, D), lambda i: (i, 0)),
            pl.BlockSpec((BM_N, 2 * D), lambda i: (i, 0)),
        ],
        out_shape=[
            jax.ShapeDtypeStruct((N, D), jnp.float32),
            jax.ShapeDtypeStruct((N, 2 * D), jnp.float32),
        ],
    )(x, wqkv, bqkv)


def _edge_stage(edge_attr, kvg, qg, wep, bep, awmat, rep, woe, boe):
    grid = (E // BM_E,)

    def body(ea_ref, kv_ref, q_ref, wep_ref, bep_ref, aw_ref, rep_ref,
             woe_ref, boe_ref,
             vplo_ref, vphi_ref, etplo_ref, etphi_ref, p128_ref,
             ee_ref, se_ref):
        i = pl.program_id(0)
        ea = ea_ref[...]
        e = (jnp.dot(ea, wep_ref[...], preferred_element_type=jnp.float32)
             + bep_ref[...])
        kv = kv_ref[...]
        kk = kv[:, :D]
        vv = kv[:, D:]
        ew = e[:, :D]
        eb = e[:, D:]
        sc = (kk + q_ref[...]) * ew
        sc = jnp.sign(sc) * jnp.sqrt(jnp.abs(sc)) + eb
        et = jnp.maximum(sc, 0.0)
        s = jnp.dot(et, aw_ref[...], preferred_element_type=jnp.float32)
        s = jnp.clip(s, -5.0, 5.0)
        p = jnp.exp(s - 5.0)
        p256 = jnp.dot(p, rep_ref[...], preferred_element_type=jnp.float32)
        vp = vv * p256
        etp = et * p256
        vplo_ref[...] = vp[:, :128]
        vphi_ref[...] = vp[:, 128:]
        etplo_ref[...] = etp[:, :128]
        etphi_ref[...] = etp[:, 128:]
        p128_ref[...] = jnp.concatenate(
            [p, jnp.ones_like(p), jnp.zeros((p.shape[0], 112), jnp.float32)],
            axis=1)
        ee = (ea
              + jnp.dot(et, woe_ref[...], preferred_element_type=jnp.float32)
              + boe_ref[...])
        ee_ref[...] = ee

        @pl.when(i == 0)
        def _():
            se_ref[...] = jnp.zeros_like(se_ref)

        se_ref[...] += jnp.concatenate(
            [jnp.sum(ee, axis=0, keepdims=True),
             jnp.sum(ee * ee, axis=0, keepdims=True),
             jnp.zeros((6, D), jnp.float32)], axis=0)

    outs = pl.pallas_call(
        body,
        grid=grid,
        in_specs=[
            pl.BlockSpec((BM_E, D), lambda i: (i, 0)),
            pl.BlockSpec((BM_E, 2 * D), lambda i: (i, 0)),
            pl.BlockSpec((BM_E, D), lambda i: (i, 0)),
            pl.BlockSpec((D, 2 * D), lambda i: (0, 0)),
            pl.BlockSpec((1, 2 * D), lambda i: (0, 0)),
            pl.BlockSpec((D, H), lambda i: (0, 0)),
            pl.BlockSpec((H, D), lambda i: (0, 0)),
            pl.BlockSpec((D, D), lambda i: (0, 0)),
            pl.BlockSpec((1, D), lambda i: (0, 0)),
        ],
        out_specs=[
            pl.BlockSpec((BM_E, 128), lambda i: (i, 0)),
            pl.BlockSpec((BM_E, 128), lambda i: (i, 0)),
            pl.BlockSpec((BM_E, 128), lambda i: (i, 0)),
            pl.BlockSpec((BM_E, 128), lambda i: (i, 0)),
            pl.BlockSpec((BM_E, 128), lambda i: (i, 0)),
            pl.BlockSpec((BM_E, D), lambda i: (i, 0)),
            pl.BlockSpec((8, D), lambda i: (0, 0)),
        ],
        out_shape=[
            jax.ShapeDtypeStruct((E, 128), jnp.float32),
            jax.ShapeDtypeStruct((E, 128), jnp.float32),
            jax.ShapeDtypeStruct((E, 128), jnp.float32),
            jax.ShapeDtypeStruct((E, 128), jnp.float32),
            jax.ShapeDtypeStruct((E, 128), jnp.float32),
            jax.ShapeDtypeStruct((E, D), jnp.float32),
            jax.ShapeDtypeStruct((8, D), jnp.float32),
        ],
    )(edge_attr, kvg, qg, wep, bep, awmat, rep, woe, boe)
    return outs


def _node_combine(vplo, vphi, etplo, etphi, p16, x, rep, vem, cc, woh, boh):
    grid = (N // BM_N,)

    def body(vplo_ref, vphi_ref, etplo_ref, etphi_ref, p16_ref, x_ref,
             rep_ref, vem_ref, cc_ref, woh_ref, boh_ref, t_ref):
        a = jnp.concatenate([vplo_ref[0] + vplo_ref[1],
                             vphi_ref[0] + vphi_ref[1]], axis=1)
        bm = jnp.concatenate([etplo_ref[0] + etplo_ref[1],
                              etphi_ref[0] + etphi_ref[1]], axis=1)
        pp = p16_ref[0] + p16_ref[1]
        ssum = pp[:, :H]
        deg = pp[:, H:H + 1]
        denom = jnp.dot(ssum, rep_ref[...],
                        preferred_element_type=jnp.float32) + 1e-16
        wv = a / denom
        rowv = jnp.dot(bm / denom, vem_ref[...],
                       preferred_element_type=jnp.float32)
        h1 = wv + rowv
        ld = jnp.log(deg + 1.0)
        h2 = h1 * cc_ref[0:1, :] + (h1 * ld) * cc_ref[1:2, :]
        t_ref[...] = (x_ref[...]
                      + jnp.dot(h2, woh_ref[...],
                                preferred_element_type=jnp.float32)
                      + boh_ref[...])

    return pl.pallas_call(
        body,
        grid=grid,
        in_specs=[
            pl.BlockSpec((2, BM_N, 128), lambda i: (0, i, 0)),
            pl.BlockSpec((2, BM_N, 128), lambda i: (0, i, 0)),
            pl.BlockSpec((2, BM_N, 128), lambda i: (0, i, 0)),
            pl.BlockSpec((2, BM_N, 128), lambda i: (0, i, 0)),
            pl.BlockSpec((2, BM_N, 128), lambda i: (0, i, 0)),
            pl.BlockSpec((BM_N, D), lambda i: (i, 0)),
            pl.BlockSpec((H, D), lambda i: (0, 0)),
            pl.BlockSpec((D, D), lambda i: (0, 0)),
            pl.BlockSpec((2, D), lambda i: (0, 0)),
            pl.BlockSpec((D, D), lambda i: (0, 0)),
            pl.BlockSpec((1, D), lambda i: (0, 0)),
        ],
        out_specs=pl.BlockSpec((BM_N, D), lambda i: (i, 0)),
        out_shape=jax.ShapeDtypeStruct((N, D), jnp.float32),
    )(vplo, vphi, etplo, etphi, p16, x, rep, vem, cc, woh, boh)


def _bn_reduce(x, bm):
    m, cols = x.shape
    grid = (m // bm,)

    def body(x_ref, s_ref):
        i = pl.program_id(0)

        @pl.when(i == 0)
        def _():
            s_ref[...] = jnp.zeros_like(s_ref)

        xv = x_ref[...]
        upd = jnp.concatenate(
            [jnp.sum(xv, axis=0, keepdims=True),
             jnp.sum(xv * xv, axis=0, keepdims=True),
             jnp.zeros((6, cols), jnp.float32)], axis=0)
        s_ref[...] += upd

    return pl.pallas_call(
        body,
        grid=grid,
        in_specs=[pl.BlockSpec((bm, cols), lambda i: (i, 0))],
        out_specs=pl.BlockSpec((8, cols), lambda i: (0, 0)),
        out_shape=jax.ShapeDtypeStruct((8, cols), jnp.float32),
    )(x)


def _bn_apply(x, stats, gb, bm, nrows):
    m, cols = x.shape
    grid = (m // bm,)
    inv_n = 1.0 / nrows

    def body(x_ref, s_ref, gb_ref, o_ref):
        s = s_ref[...]
        mu = s[0:1, :] * inv_n
        var = s[1:2, :] * inv_n - mu * mu
        inv = lax.rsqrt(var + 1e-5)
        o_ref[...] = gb_ref[0:1, :] * (x_ref[...] - mu) * inv + gb_ref[1:2, :]

    return pl.pallas_call(
        body,
        grid=grid,
        in_specs=[
            pl.BlockSpec((bm, cols), lambda i: (i, 0)),
            pl.BlockSpec((8, cols), lambda i: (0, 0)),
            pl.BlockSpec((2, cols), lambda i: (0, 0)),
        ],
        out_specs=pl.BlockSpec((bm, cols), lambda i: (i, 0)),
        out_shape=jax.ShapeDtypeStruct((m, cols), jnp.float32),
    )(x, stats, gb)


def _ffn_stage(t, stats, gb1, w1, b1, w2, b2):
    grid = (N // BM_N,)
    inv_n = 1.0 / N

    def body(t_ref, s_ref, gb_ref, w1_ref, b1_ref, w2_ref, b2_ref,
             h3_ref, s2_ref):
        i = pl.program_id(0)
        s = s_ref[...]
        mu = s[0:1, :] * inv_n
        var = s[1:2, :] * inv_n - mu * mu
        inv = lax.rsqrt(var + 1e-5)
        hb = gb_ref[0:1, :] * (t_ref[...] - mu) * inv + gb_ref[1:2, :]
        f = jnp.maximum(
            jnp.dot(hb, w1_ref[...], preferred_element_type=jnp.float32)
            + b1_ref[...], 0.0)
        h3 = hb + (jnp.dot(f, w2_ref[...], preferred_element_type=jnp.float32)
                   + b2_ref[...])
        h3_ref[...] = h3

        @pl.when(i == 0)
        def _():
            s2_ref[...] = jnp.zeros_like(s2_ref)

        s2_ref[...] += jnp.concatenate(
            [jnp.sum(h3, axis=0, keepdims=True),
             jnp.sum(h3 * h3, axis=0, keepdims=True),
             jnp.zeros((6, D), jnp.float32)], axis=0)

    return pl.pallas_call(
        body,
        grid=grid,
        in_specs=[
            pl.BlockSpec((BM_N, D), lambda i: (i, 0)),
            pl.BlockSpec((8, D), lambda i: (0, 0)),
            pl.BlockSpec((2, D), lambda i: (0, 0)),
            pl.BlockSpec((D, 2 * D), lambda i: (0, 0)),
            pl.BlockSpec((1, 2 * D), lambda i: (0, 0)),
            pl.BlockSpec((2 * D, D), lambda i: (0, 0)),
            pl.BlockSpec((1, D), lambda i: (0, 0)),
        ],
        out_specs=[
            pl.BlockSpec((BM_N, D), lambda i: (i, 0)),
            pl.BlockSpec((8, D), lambda i: (0, 0)),
        ],
        out_shape=[
            jax.ShapeDtypeStruct((N, D), jnp.float32),
            jax.ShapeDtypeStruct((8, D), jnp.float32),
        ],
    )(t, stats, gb1, w1, b1, w2, b2)


_WE_PERM = np.array(
    [h * 2 * DH + j for h in range(H) for j in range(DH)]
    + [h * 2 * DH + DH + j for h in range(H) for j in range(DH)],
    dtype=np.int32)

_REP = np.zeros((H, D), np.float32)
for _h in range(H):
    _REP[_h, _h * DH:(_h + 1) * DH] = 1.0

_HEAD_MASK = (np.arange(D)[:, None] // DH == np.arange(H)[None, :])


def kernel(x, edge_index, edge_attr, params):
    src = edge_index[0]
    dst = edge_index[1]

    wqkv = jnp.concatenate([params['Wq'], params['Wk'], params['Wv']], axis=1)
    bqkv = jnp.concatenate(
        [params['bq'], jnp.zeros((2 * H * DH,), jnp.float32)])[None, :]
    wep = params['We'][:, _WE_PERM]
    bep = params['be'][_WE_PERM][None, :]
    rep = jnp.asarray(_REP)
    aw2 = params['Aw'][:, :, 0]
    awmat = jnp.where(jnp.asarray(_HEAD_MASK),
                      jnp.tile(aw2, (H, 1)), 0.0)
    vem = jax.scipy.linalg.block_diag(
        *[params['VeRow'][:, h, :] for h in range(H)])
    cc = params['deg_coef'][0].T
    gb1h = jnp.stack([params['g1h'], params['b1h']])
    gb1e = jnp.stack([params['g1e'], params['b1e']])
    gb2h = jnp.stack([params['g2h'], params['b2h']])
    zeros128 = jnp.zeros((NPAD // NS, 128), jnp.float32)

    qt, kvt = _qkv_proj(x, wqkv, bqkv)

    kvg = _gather_rows(kvt, src, 2 * D)
    qg = _gather_rows(qt, dst, D)

    vplo, vphi, etplo, etphi, p16, ee_pre, stats_e = _edge_stage(
        edge_attr, kvg, qg, wep, bep, awmat, rep,
        params['Woe'], params['boe'][None, :])

    a_lo = _scatter_add(vplo, dst, zeros128, 128)[:, :N, :]
    a_hi = _scatter_add(vphi, dst, zeros128, 128)[:, :N, :]
    b_lo = _scatter_add(etplo, dst, zeros128, 128)[:, :N, :]
    b_hi = _scatter_add(etphi, dst, zeros128, 128)[:, :N, :]
    pacc = _scatter_add(p16, dst, zeros128, 128)[:, :N, :]

    t = _node_combine(a_lo, a_hi, b_lo, b_hi, pacc, x, rep, vem, cc,
                      params['Woh'], params['boh'][None, :])

    stats1 = _bn_reduce(t, BM_N)
    h3, stats2 = _ffn_stage(t, stats1, gb1h, params['W1'],
                            params['bf1'][None, :], params['W2'],
                            params['bf2'][None, :])
    h_out = _bn_apply(h3, stats2, gb2h, BM_N, N)

    ee_out = _bn_apply(ee_pre, stats_e, gb1e, BM_E, E)

    return h_out, ee_out


# speedup vs baseline: 30.6162x; 1.1527x over previous
import functools

import jax
import jax.numpy as jnp
import numpy as np
from jax import lax
from jax.experimental import pallas as pl
from jax.experimental.pallas import tpu as pltpu
from jax.experimental.pallas import tpu_sc as plsc

N = 10000
E = 160000
D = 256
H = 8
DH = 32

NPAD = 10240
NC = 2
NS = 16
NW = NC * NS
SCCH = 128

BM_E = 1000
BM_N = 1000


def _gather_rows(table, idx, dt):
    etot = idx.shape[0]
    nchunks = etot // SCCH
    nfull = nchunks // NW
    rem = nchunks % NW
    mesh = plsc.VectorSubcoreMesh(core_axis_name="c", subcore_axis_name="s")

    @functools.partial(
        pl.kernel,
        mesh=mesh,
        out_type=jax.ShapeDtypeStruct((etot, dt), jnp.float32),
        scratch_types=[
            pltpu.VMEM((SCCH,), jnp.int32),
            pltpu.VMEM((SCCH, dt), jnp.float32),
            pltpu.SemaphoreType.DMA,
        ],
    )
    def k(table_hbm, idx_hbm, out_hbm, idx_v, rows_v, sem):
        wid = lax.axis_index("s") * NC + lax.axis_index("c")

        def body(j, carry):
            base = (j * NW + wid) * SCCH
            pltpu.sync_copy(idx_hbm.at[pl.ds(base, SCCH)], idx_v)
            pltpu.async_copy(table_hbm.at[idx_v], rows_v, sem).wait()
            pltpu.sync_copy(rows_v, out_hbm.at[pl.ds(base, SCCH)])
            return carry

        lax.fori_loop(0, nfull, body, 0)

        @pl.when(wid < rem)
        def _():
            body(nfull, 0)

    return k(table, idx)


def _scatter_all(vals_list, idx2d, zeros):
    nv = len(vals_list)
    rows_per_tile = NPAD // NS
    cpt = (E // SCCH) // NW
    nrem = (E // SCCH) % NW
    mesh = plsc.VectorSubcoreMesh(core_axis_name="c", subcore_axis_name="s")

    @functools.partial(
        pl.kernel,
        mesh=mesh,
        out_type=jax.ShapeDtypeStruct((nv * NC * NPAD, 128), jnp.float32),
        scratch_types=[
            pltpu.VMEM((cpt + 1, 1, SCCH), jnp.int32),
            pltpu.VMEM((2, SCCH, 128), jnp.float32),
            pltpu.VMEM_SHARED((NPAD, 128), jnp.float32),
            pltpu.SemaphoreType.DMA,
            pltpu.SemaphoreType.DMA,
        ],
    )
    def k(*refs):
        vrefs = list(refs[:nv])
        idx_hbm, zeros_hbm, out_hbm, idx_v, val_v, acc = refs[nv:nv + 6]
        s0, s1 = refs[nv + 6:nv + 8]
        ci = lax.axis_index("c")
        si = lax.axis_index("s")
        wid = si * NC + ci
        r0 = si * rows_per_tile
        row0 = wid * cpt
        pltpu.sync_copy(idx_hbm.at[pl.ds(row0, cpt)], idx_v.at[pl.ds(0, cpt)])

        @pl.when(wid < nrem)
        def _():
            pltpu.sync_copy(idx_hbm.at[pl.ds(NW * cpt + wid, 1)],
                            idx_v.at[pl.ds(cpt, 1)])
        for ph in range(nv):
            vh = vrefs[ph]
            pltpu.sync_copy(zeros_hbm, acc.at[pl.ds(r0, rows_per_tile)])
            plsc.subcore_barrier()

            def load(jj, b, sem):
                pltpu.async_copy(vh.at[pl.ds((row0 + jj) * SCCH, SCCH)],
                                 val_v.at[b], sem)

            def wait(b, sem):
                pltpu.make_async_copy(vh.at[pl.ds(0, SCCH)],
                                      val_v.at[b], sem).wait()

            def scat(jj, b):
                pltpu.sync_copy(val_v.at[b], acc.at[idx_v.at[jj, 0]], add=True)

            npair = cpt // 2
            load(0, 0, s0)

            def body(g, carry):
                j = 2 * g
                load(j + 1, 1, s1)
                wait(0, s0)
                scat(j, 0)

                @pl.when(g < npair - 1)
                def _():
                    load(j + 2, 0, s0)

                wait(1, s1)
                scat(j + 1, 1)
                return carry

            lax.fori_loop(0, npair, body, 0)
            for jj in range(2 * npair, cpt):
                pltpu.sync_copy(vh.at[pl.ds((row0 + jj) * SCCH, SCCH)],
                                val_v.at[0])
                scat(jj, 0)

            @pl.when(wid < nrem)
            def _():
                pltpu.sync_copy(vh.at[pl.ds((NW * cpt + wid) * SCCH, SCCH)],
                                val_v.at[0])
                scat(cpt, 0)

            plsc.subcore_barrier()
            pltpu.sync_copy(
                acc.at[pl.ds(r0, rows_per_tile)],
                out_hbm.at[pl.ds((ph * NC + ci) * NPAD + r0, rows_per_tile)])
            plsc.subcore_barrier()

    return k(*vals_list, idx2d, zeros)


def _mm_bias(x, w, b, bm, extra=None):
    m, kdim = x.shape
    nc = w.shape[1]
    grid = (m // bm,)

    def body_noextra(x_ref, w_ref, b_ref, o_ref):
        o_ref[...] = (jnp.dot(x_ref[...], w_ref[...],
                              preferred_element_type=jnp.float32) + b_ref[...])

    def body_extra(x_ref, w_ref, b_ref, e_ref, o_ref):
        o_ref[...] = (jnp.dot(x_ref[...], w_ref[...],
                              preferred_element_type=jnp.float32)
                      + b_ref[...] + e_ref[...])

    in_specs = [
        pl.BlockSpec((bm, kdim), lambda i: (i, 0)),
        pl.BlockSpec((kdim, nc), lambda i: (0, 0)),
        pl.BlockSpec((1, nc), lambda i: (0, 0)),
    ]
    args = [x, w, b]
    body = body_noextra
    if extra is not None:
        in_specs.append(pl.BlockSpec((bm, nc), lambda i: (i, 0)))
        args.append(extra)
        body = body_extra
    return pl.pallas_call(
        body,
        grid=grid,
        in_specs=in_specs,
        out_specs=pl.BlockSpec((bm, nc), lambda i: (i, 0)),
        out_shape=jax.ShapeDtypeStruct((m, nc), jnp.float32),
    )(*args)


def _qkv_proj(x, wqkv, bqkv):
    grid = (N // BM_N,)

    def body(x_ref, w_ref, b_ref, q_ref, kv_ref):
        y = (jnp.dot(x_ref[...], w_ref[...],
                     preferred_element_type=jnp.float32) + b_ref[...])
        q_ref[...] = y[:, :D]
        kv_ref[...] = y[:, D:]

    return pl.pallas_call(
        body,
        grid=grid,
        in_specs=[
            pl.BlockSpec((BM_N, D), lambda i: (i, 0)),
            pl.BlockSpec((D, 3 * D), lambda i: (0, 0)),
            pl.BlockSpec((1, 3 * D), lambda i: (0, 0)),
        ],
        out_specs=[
            pl.BlockSpec((BM_N, D), lambda i: (i, 0)),
            pl.BlockSpec((BM_N, 2 * D), lambda i: (i, 0)),
        ],
        out_shape=[
            jax.ShapeDtypeStruct((N, D), jnp.float32),
            jax.ShapeDtypeStruct((N, 2 * D), jnp.float32),
        ],
    )(x, wqkv, bqkv)


def _edge_stage(edge_attr, kvg, qg, wep, bep, awmat, rep, woe, boe):
    grid = (E // BM_E,)

    def body(ea_ref, kv_ref, q_ref, wep_ref, bep_ref, aw_ref, rep_ref,
             woe_ref, boe_ref,
             vplo_ref, vphi_ref, etplo_ref, etphi_ref, p128_ref,
             ee_ref, se_ref):
        i = pl.program_id(0)
        ea = ea_ref[...]
        e = (jnp.dot(ea, wep_ref[...], preferred_element_type=jnp.float32)
             + bep_ref[...])
        kv = kv_ref[...]
        kk = kv[:, :D]
        vv = kv[:, D:]
        ew = e[:, :D]
        eb = e[:, D:]
        sc = (kk + q_ref[...]) * ew
        sc = jnp.sign(sc) * jnp.sqrt(jnp.abs(sc)) + eb
        et = jnp.maximum(sc, 0.0)
        s = jnp.dot(et, aw_ref[...], preferred_element_type=jnp.float32)
        s = jnp.clip(s, -5.0, 5.0)
        p = jnp.exp(s - 5.0)
        p256 = jnp.dot(p, rep_ref[...], preferred_element_type=jnp.float32)
        vp = vv * p256
        etp = et * p256
        vplo_ref[...] = vp[:, :128]
        vphi_ref[...] = vp[:, 128:]
        etplo_ref[...] = etp[:, :128]
        etphi_ref[...] = etp[:, 128:]
        p128_ref[...] = jnp.concatenate(
            [p, jnp.ones_like(p), jnp.zeros((p.shape[0], 112), jnp.float32)],
            axis=1)
        ee = (ea
              + jnp.dot(et, woe_ref[...], preferred_element_type=jnp.float32)
              + boe_ref[...])
        ee_ref[...] = ee

        @pl.when(i == 0)
        def _():
            se_ref[...] = jnp.zeros_like(se_ref)

        se_ref[...] += jnp.concatenate(
            [jnp.sum(ee, axis=0, keepdims=True),
             jnp.sum(ee * ee, axis=0, keepdims=True),
             jnp.zeros((6, D), jnp.float32)], axis=0)

    outs = pl.pallas_call(
        body,
        grid=grid,
        in_specs=[
            pl.BlockSpec((BM_E, D), lambda i: (i, 0)),
            pl.BlockSpec((BM_E, 2 * D), lambda i: (i, 0)),
            pl.BlockSpec((BM_E, D), lambda i: (i, 0)),
            pl.BlockSpec((D, 2 * D), lambda i: (0, 0)),
            pl.BlockSpec((1, 2 * D), lambda i: (0, 0)),
            pl.BlockSpec((D, H), lambda i: (0, 0)),
            pl.BlockSpec((H, D), lambda i: (0, 0)),
            pl.BlockSpec((D, D), lambda i: (0, 0)),
            pl.BlockSpec((1, D), lambda i: (0, 0)),
        ],
        out_specs=[
            pl.BlockSpec((BM_E, 128), lambda i: (i, 0)),
            pl.BlockSpec((BM_E, 128), lambda i: (i, 0)),
            pl.BlockSpec((BM_E, 128), lambda i: (i, 0)),
            pl.BlockSpec((BM_E, 128), lambda i: (i, 0)),
            pl.BlockSpec((BM_E, 128), lambda i: (i, 0)),
            pl.BlockSpec((BM_E, D), lambda i: (i, 0)),
            pl.BlockSpec((8, D), lambda i: (0, 0)),
        ],
        out_shape=[
            jax.ShapeDtypeStruct((E, 128), jnp.float32),
            jax.ShapeDtypeStruct((E, 128), jnp.float32),
            jax.ShapeDtypeStruct((E, 128), jnp.float32),
            jax.ShapeDtypeStruct((E, 128), jnp.float32),
            jax.ShapeDtypeStruct((E, 128), jnp.float32),
            jax.ShapeDtypeStruct((E, D), jnp.float32),
            jax.ShapeDtypeStruct((8, D), jnp.float32),
        ],
    )(edge_attr, kvg, qg, wep, bep, awmat, rep, woe, boe)
    return outs


def _node_combine(vplo, vphi, etplo, etphi, p16, x, rep, vem, cc, woh, boh):
    grid = (N // BM_N,)

    def body(vplo_ref, vphi_ref, etplo_ref, etphi_ref, p16_ref, x_ref,
             rep_ref, vem_ref, cc_ref, woh_ref, boh_ref, t_ref):
        a = jnp.concatenate([vplo_ref[0] + vplo_ref[1],
                             vphi_ref[0] + vphi_ref[1]], axis=1)
        bm = jnp.concatenate([etplo_ref[0] + etplo_ref[1],
                              etphi_ref[0] + etphi_ref[1]], axis=1)
        pp = p16_ref[0] + p16_ref[1]
        ssum = pp[:, :H]
        deg = pp[:, H:H + 1]
        denom = jnp.dot(ssum, rep_ref[...],
                        preferred_element_type=jnp.float32) + 1e-16
        wv = a / denom
        rowv = jnp.dot(bm / denom, vem_ref[...],
                       preferred_element_type=jnp.float32)
        h1 = wv + rowv
        ld = jnp.log(deg + 1.0)
        h2 = h1 * cc_ref[0:1, :] + (h1 * ld) * cc_ref[1:2, :]
        t_ref[...] = (x_ref[...]
                      + jnp.dot(h2, woh_ref[...],
                                preferred_element_type=jnp.float32)
                      + boh_ref[...])

    return pl.pallas_call(
        body,
        grid=grid,
        in_specs=[
            pl.BlockSpec((2, BM_N, 128), lambda i: (0, i, 0)),
            pl.BlockSpec((2, BM_N, 128), lambda i: (0, i, 0)),
            pl.BlockSpec((2, BM_N, 128), lambda i: (0, i, 0)),
            pl.BlockSpec((2, BM_N, 128), lambda i: (0, i, 0)),
            pl.BlockSpec((2, BM_N, 128), lambda i: (0, i, 0)),
            pl.BlockSpec((BM_N, D), lambda i: (i, 0)),
            pl.BlockSpec((H, D), lambda i: (0, 0)),
            pl.BlockSpec((D, D), lambda i: (0, 0)),
            pl.BlockSpec((2, D), lambda i: (0, 0)),
            pl.BlockSpec((D, D), lambda i: (0, 0)),
            pl.BlockSpec((1, D), lambda i: (0, 0)),
        ],
        out_specs=pl.BlockSpec((BM_N, D), lambda i: (i, 0)),
        out_shape=jax.ShapeDtypeStruct((N, D), jnp.float32),
    )(vplo, vphi, etplo, etphi, p16, x, rep, vem, cc, woh, boh)


def _bn_reduce(x, bm):
    m, cols = x.shape
    grid = (m // bm,)

    def body(x_ref, s_ref):
        i = pl.program_id(0)

        @pl.when(i == 0)
        def _():
            s_ref[...] = jnp.zeros_like(s_ref)

        xv = x_ref[...]
        upd = jnp.concatenate(
            [jnp.sum(xv, axis=0, keepdims=True),
             jnp.sum(xv * xv, axis=0, keepdims=True),
             jnp.zeros((6, cols), jnp.float32)], axis=0)
        s_ref[...] += upd

    return pl.pallas_call(
        body,
        grid=grid,
        in_specs=[pl.BlockSpec((bm, cols), lambda i: (i, 0))],
        out_specs=pl.BlockSpec((8, cols), lambda i: (0, 0)),
        out_shape=jax.ShapeDtypeStruct((8, cols), jnp.float32),
    )(x)


def _bn_apply(x, stats, gb, bm, nrows):
    m, cols = x.shape
    grid = (m // bm,)
    inv_n = 1.0 / nrows

    def body(x_ref, s_ref, gb_ref, o_ref):
        s = s_ref[...]
        mu = s[0:1, :] * inv_n
        var = s[1:2, :] * inv_n - mu * mu
        inv = lax.rsqrt(var + 1e-5)
        o_ref[...] = gb_ref[0:1, :] * (x_ref[...] - mu) * inv + gb_ref[1:2, :]

    return pl.pallas_call(
        body,
        grid=grid,
        in_specs=[
            pl.BlockSpec((bm, cols), lambda i: (i, 0)),
            pl.BlockSpec((8, cols), lambda i: (0, 0)),
            pl.BlockSpec((2, cols), lambda i: (0, 0)),
        ],
        out_specs=pl.BlockSpec((bm, cols), lambda i: (i, 0)),
        out_shape=jax.ShapeDtypeStruct((m, cols), jnp.float32),
    )(x, stats, gb)


def _ffn_stage(t, stats, gb1, w1, b1, w2, b2):
    grid = (N // BM_N,)
    inv_n = 1.0 / N

    def body(t_ref, s_ref, gb_ref, w1_ref, b1_ref, w2_ref, b2_ref,
             h3_ref, s2_ref):
        i = pl.program_id(0)
        s = s_ref[...]
        mu = s[0:1, :] * inv_n
        var = s[1:2, :] * inv_n - mu * mu
        inv = lax.rsqrt(var + 1e-5)
        hb = gb_ref[0:1, :] * (t_ref[...] - mu) * inv + gb_ref[1:2, :]
        f = jnp.maximum(
            jnp.dot(hb, w1_ref[...], preferred_element_type=jnp.float32)
            + b1_ref[...], 0.0)
        h3 = hb + (jnp.dot(f, w2_ref[...], preferred_element_type=jnp.float32)
                   + b2_ref[...])
        h3_ref[...] = h3

        @pl.when(i == 0)
        def _():
            s2_ref[...] = jnp.zeros_like(s2_ref)

        s2_ref[...] += jnp.concatenate(
            [jnp.sum(h3, axis=0, keepdims=True),
             jnp.sum(h3 * h3, axis=0, keepdims=True),
             jnp.zeros((6, D), jnp.float32)], axis=0)

    return pl.pallas_call(
        body,
        grid=grid,
        in_specs=[
            pl.BlockSpec((BM_N, D), lambda i: (i, 0)),
            pl.BlockSpec((8, D), lambda i: (0, 0)),
            pl.BlockSpec((2, D), lambda i: (0, 0)),
            pl.BlockSpec((D, 2 * D), lambda i: (0, 0)),
            pl.BlockSpec((1, 2 * D), lambda i: (0, 0)),
            pl.BlockSpec((2 * D, D), lambda i: (0, 0)),
            pl.BlockSpec((1, D), lambda i: (0, 0)),
        ],
        out_specs=[
            pl.BlockSpec((BM_N, D), lambda i: (i, 0)),
            pl.BlockSpec((8, D), lambda i: (0, 0)),
        ],
        out_shape=[
            jax.ShapeDtypeStruct((N, D), jnp.float32),
            jax.ShapeDtypeStruct((8, D), jnp.float32),
        ],
    )(t, stats, gb1, w1, b1, w2, b2)


_WE_PERM = np.array(
    [h * 2 * DH + j for h in range(H) for j in range(DH)]
    + [h * 2 * DH + DH + j for h in range(H) for j in range(DH)],
    dtype=np.int32)

_REP = np.zeros((H, D), np.float32)
for _h in range(H):
    _REP[_h, _h * DH:(_h + 1) * DH] = 1.0

_HEAD_MASK = (np.arange(D)[:, None] // DH == np.arange(H)[None, :])


def kernel(x, edge_index, edge_attr, params):
    src = edge_index[0]
    dst = edge_index[1]

    wqkv = jnp.concatenate([params['Wq'], params['Wk'], params['Wv']], axis=1)
    bqkv = jnp.concatenate(
        [params['bq'], jnp.zeros((2 * H * DH,), jnp.float32)])[None, :]
    wep = params['We'][:, _WE_PERM]
    bep = params['be'][_WE_PERM][None, :]
    rep = jnp.asarray(_REP)
    aw2 = params['Aw'][:, :, 0]
    awmat = jnp.where(jnp.asarray(_HEAD_MASK),
                      jnp.tile(aw2, (H, 1)), 0.0)
    vem = jax.scipy.linalg.block_diag(
        *[params['VeRow'][:, h, :] for h in range(H)])
    cc = params['deg_coef'][0].T
    gb1h = jnp.stack([params['g1h'], params['b1h']])
    gb1e = jnp.stack([params['g1e'], params['b1e']])
    gb2h = jnp.stack([params['g2h'], params['b2h']])
    zeros128 = jnp.zeros((NPAD // NS, 128), jnp.float32)

    qt, kvt = _qkv_proj(x, wqkv, bqkv)

    kvg = _gather_rows(kvt, src, 2 * D)
    qg = _gather_rows(qt, dst, D)

    vplo, vphi, etplo, etphi, p16, ee_pre, stats_e = _edge_stage(
        edge_attr, kvg, qg, wep, bep, awmat, rep,
        params['Woe'], params['boe'][None, :])

    idx2d = dst.reshape(E // SCCH, 1, SCCH)
    scat = _scatter_all([vplo, vphi, etplo, etphi, p16], idx2d, zeros128)
    scat = scat.reshape(5, NC, NPAD, 128)
    a_lo = scat[0][:, :N, :]
    a_hi = scat[1][:, :N, :]
    b_lo = scat[2][:, :N, :]
    b_hi = scat[3][:, :N, :]
    pacc = scat[4][:, :N, :]

    t = _node_combine(a_lo, a_hi, b_lo, b_hi, pacc, x, rep, vem, cc,
                      params['Woh'], params['boh'][None, :])

    stats1 = _bn_reduce(t, BM_N)
    h3, stats2 = _ffn_stage(t, stats1, gb1h, params['W1'],
                            params['bf1'][None, :], params['W2'],
                            params['bf2'][None, :])
    h_out = _bn_apply(h3, stats2, gb2h, BM_N, N)

    ee_out = _bn_apply(ee_pre, stats_e, gb1e, BM_E, E)

    return h_out, ee_out
